# baseline (speedup 1.0000x reference)
import dataclasses
import functools

import jax
import jax.numpy as jnp
from jax import lax
from jax.experimental import pallas as pl
from jax.experimental.pallas import tpu as pltpu
from jax.experimental.pallas import tpu_sc as plsc

N = 10000
D = 128
OUT = 32
NNZ = 320000
ROWS = NNZ // 128
NC = 2
NS = 16
NW = NC * NS
FULL_T = ROWS // NW
REM = ROWS - FULL_T * NW
ND = N * D
SL1 = ND // NS

_mesh = plsc.VectorSubcoreMesh(core_axis_name="c", subcore_axis_name="s")

_cp = pltpu.CompilerParams()
for _f, _v in (("needs_layout_passes", False), ("use_tc_tiling_on_sc", False)):
    if _f in pltpu.CompilerParams.__dataclass_fields__:
        _cp = dataclasses.replace(_cp, **{_f: _v})


W1 = 512
ROWS1 = NNZ // W1
FT1 = ROWS1 // NW
REM1 = ROWS1 - FT1 * NW


def _worker_ids():
    cid = lax.axis_index("c")
    sid = lax.axis_index("s")
    gwid = sid * NC + cid
    return cid, sid, gwid


def _bulk_load(pairs, cbase, gwid, ft, rem, sems):
    @pl.when(gwid < rem)
    def _():
        ds = [pltpu.async_copy(hbm.at[pl.ds(cbase, ft + 1)], vmem, sem)
              for (hbm, vmem), sem in zip(pairs, sems)]
        for d in ds:
            d.wait()

    @pl.when(gwid >= rem)
    def _():
        ds = [pltpu.async_copy(hbm.at[pl.ds(cbase, ft)],
                               vmem.at[pl.ds(0, ft)], sem)
              for (hbm, vmem), sem in zip(pairs, sems)]
        for d in ds:
            d.wait()


@functools.partial(
    pl.kernel,
    mesh=_mesh,
    compiler_params=_cp,
    out_type=jax.ShapeDtypeStruct((NC * ND,), jnp.float32),
    scratch_types=[
        pltpu.VMEM((FT1 + 1, W1), jnp.int32),
        pltpu.VMEM((FT1 + 1, W1), jnp.float32),
        pltpu.VMEM_SHARED((ND,), jnp.float32),
        pltpu.SemaphoreType.DMA,
        pltpu.SemaphoreType.DMA,
        pltpu.SemaphoreType.DMA,
    ],
)
def _stage1(xf_hbm, xv_hbm, z_hbm, out_hbm, idx_all, vals_all, xd_sh,
            s0, s1, s2):
    cid, sid, gwid = _worker_ids()
    cbase = gwid * FT1 + jnp.minimum(gwid, REM1)

    dz = pltpu.async_copy(z_hbm, xd_sh.at[pl.ds(sid * SL1, SL1)], s1)
    _bulk_load([(xf_hbm, idx_all), (xv_hbm, vals_all)], cbase, gwid,
               FT1, REM1, (s0, s2))
    dz.wait()
    plsc.subcore_barrier()

    @pl.loop(0, FT1)
    def _(t):
        pltpu.sync_copy(vals_all.at[t], xd_sh.at[idx_all.at[t]], add=True)

    @pl.when(gwid < REM1)
    def _():
        pltpu.sync_copy(vals_all.at[FT1], xd_sh.at[idx_all.at[FT1]],
                        add=True)

    plsc.subcore_barrier()
    pltpu.sync_copy(xd_sh.at[pl.ds(sid * SL1, SL1)],
                    out_hbm.at[pl.ds(cid * ND + sid * SL1, SL1)])


@functools.partial(
    pl.kernel,
    mesh=_mesh,
    compiler_params=_cp,
    out_type=jax.ShapeDtypeStruct((NC * N, OUT), jnp.float32),
    scratch_types=[
        pltpu.VMEM((FULL_T + 1, 128), jnp.int32),
        pltpu.VMEM((FULL_T + 1, 128), jnp.int32),
        pltpu.VMEM((FULL_T + 1, 128), jnp.float32),
        pltpu.VMEM((3, 128, OUT), jnp.float32),
        pltpu.VMEM_SHARED((N, OUT), jnp.float32),
        pltpu.SemaphoreType.DMA,
        pltpu.SemaphoreType.DMA,
        pltpu.SemaphoreType.DMA,
        pltpu.SemaphoreType.DMA,
        pltpu.SemaphoreType.DMA,
        pltpu.SemaphoreType.DMA,
    ],
)
def _stage2(ar_hbm, ac_hbm, av_hbm, h_hbm, z_hbm, out_hbm,
            rows_all, cols_all, vals_all, bufs, y_sh,
            g0, g1, g2, s0, s1, s2):
    cid, sid, gwid = _worker_ids()
    cbase = gwid * FULL_T + jnp.minimum(gwid, REM)

    @pl.when(sid < 15)
    def _():
        pltpu.async_copy(z_hbm.at[pl.ds(0, 624)],
                         y_sh.at[pl.ds(sid * 624, 624)], s0)

    @pl.when(sid == 15)
    def _():
        pltpu.async_copy(z_hbm, y_sh.at[pl.ds(15 * 624, 640)], s0)

    _bulk_load([(ar_hbm, rows_all), (ac_hbm, cols_all), (av_hbm, vals_all)],
               cbase, gwid, FULL_T, REM, (g0, g1, g2))

    @pl.when(sid < 15)
    def _():
        pltpu.make_async_copy(z_hbm.at[pl.ds(0, 624)],
                              y_sh.at[pl.ds(sid * 624, 624)], s0).wait()

    @pl.when(sid == 15)
    def _():
        pltpu.make_async_copy(z_hbm, y_sh.at[pl.ds(15 * 624, 640)], s0).wait()

    plsc.subcore_barrier()

    def scale(t, b):
        @pl.loop(0, 8, unroll=8)
        def _(g):
            v16 = vals_all[t, pl.ds(g * 16, 16)]
            for j in range(16):
                k = g * 16 + j
                bc = jnp.full((16,), v16[j], jnp.float32)
                bufs[b, k, pl.ds(0, 16)] = bufs[b, k, pl.ds(0, 16)] * bc
                bufs[b, k, pl.ds(16, 16)] = bufs[b, k, pl.ds(16, 16)] * bc

    nchunks = jnp.where(gwid < REM, FULL_T + 1, FULL_T)
    gsem = (g0, g1, g2)
    ssem = (s0, s1, s2)

    pltpu.async_copy(h_hbm.at[cols_all.at[0]], bufs.at[0], g0)
    pltpu.async_copy(h_hbm.at[cols_all.at[1]], bufs.at[1], g1)

    def chunk_body(c, b):
        bn = (b + 2) % 3
        pltpu.make_async_copy(h_hbm.at[cols_all.at[c]], bufs.at[b],
                              gsem[b]).wait()
        scale(c, b)

        @pl.when(c > 0)
        def _():
            pltpu.make_async_copy(bufs.at[bn], y_sh.at[rows_all.at[c - 1]],
                                  ssem[bn]).wait()

        @pl.when(c + 2 < nchunks)
        def _():
            pltpu.async_copy(h_hbm.at[cols_all.at[c + 2]], bufs.at[bn],
                             gsem[bn])

        pltpu.async_copy(bufs.at[b], y_sh.at[rows_all.at[c]], ssem[b],
                         add=True)

    @pl.loop(0, FULL_T, step=3)
    def _(t):
        chunk_body(t, 0)
        chunk_body(t + 1, 1)
        chunk_body(t + 2, 2)

    pltpu.make_async_copy(bufs.at[2], y_sh.at[rows_all.at[FULL_T - 1]],
                          ssem[2]).wait()

    @pl.when(gwid < REM)
    def _():
        pltpu.make_async_copy(h_hbm.at[cols_all.at[FULL_T]], bufs.at[0],
                              gsem[0]).wait()
        scale(FULL_T, 0)
        pltpu.sync_copy(bufs.at[0], y_sh.at[rows_all.at[FULL_T]], add=True)

    plsc.subcore_barrier()

    @pl.when(sid < 15)
    def _():
        pltpu.sync_copy(y_sh.at[pl.ds(sid * 624, 624)],
                        out_hbm.at[pl.ds(cid * N + sid * 624, 624)])

    @pl.when(sid == 15)
    def _():
        pltpu.sync_copy(y_sh.at[pl.ds(15 * 624, 640)],
                        out_hbm.at[pl.ds(cid * N + 15 * 624, 640)])


def _mm_body(x0_ref, x1_ref, w_ref, h_ref):
    bn = h_ref.shape[0]
    x = (x0_ref[...] + x1_ref[...]).reshape(bn, D)
    h_ref[...] = jnp.dot(x, w_ref[...], preferred_element_type=jnp.float32)


def _matmul(xp, w):
    bn = 2000
    nb = N // bn
    return pl.pallas_call(
        _mm_body,
        grid=(nb,),
        in_specs=[
            pl.BlockSpec((bn * D,), lambda i: (i,)),
            pl.BlockSpec((bn * D,), lambda i: (i + nb,)),
            pl.BlockSpec((D, OUT), lambda i: (0, 0)),
        ],
        out_specs=pl.BlockSpec((bn, OUT), lambda i: (i, 0)),
        out_shape=jax.ShapeDtypeStruct((N, OUT), jnp.float32),
    )(xp, xp, w)


def _fin_body(y0_ref, y1_ref, o_ref):
    o_ref[...] = jnp.maximum(y0_ref[...] + y1_ref[...], 0.0)


def _finish(yp):
    bn = 2000
    nb = N // bn
    return pl.pallas_call(
        _fin_body,
        grid=(nb,),
        in_specs=[
            pl.BlockSpec((bn, OUT), lambda i: (i, 0)),
            pl.BlockSpec((bn, OUT), lambda i: (i + nb, 0)),
        ],
        out_specs=pl.BlockSpec((bn, OUT), lambda i: (i, 0)),
        out_shape=jax.ShapeDtypeStruct((N, OUT), jnp.float32),
    )(yp, yp)


def kernel(x_rows, x_cols, x_vals, adj_rows, adj_cols, adj_vals, kernel):
    xf = (x_rows.astype(jnp.int32) * D
          + x_cols.astype(jnp.int32)).reshape(ROWS1, W1)
    xv = x_vals.reshape(ROWS1, W1)
    ar = adj_rows.astype(jnp.int32).reshape(ROWS, 128)
    ac = adj_cols.astype(jnp.int32).reshape(ROWS, 128)
    av = adj_vals.reshape(ROWS, 128)
    z1 = jnp.zeros((SL1,), jnp.float32)
    z2 = jnp.zeros((640, OUT), jnp.float32)

    xd = _stage1(xf, xv, z1)
    h = _matmul(xd, kernel)
    yp = _stage2(ar, ac, av, h, z2)
    return _finish(yp)


# speedup vs baseline: 1.0058x; 1.0058x over previous
import dataclasses
import functools

import jax
import jax.numpy as jnp
from jax import lax
from jax.experimental import pallas as pl
from jax.experimental.pallas import tpu as pltpu
from jax.experimental.pallas import tpu_sc as plsc

N = 10000
D = 128
OUT = 32
NNZ = 320000
ROWS = NNZ // 128
NC = 2
NS = 16
NW = NC * NS
FULL_T = ROWS // NW
REM = ROWS - FULL_T * NW
ND = N * D
SL1 = ND // NS

_mesh = plsc.VectorSubcoreMesh(core_axis_name="c", subcore_axis_name="s")

_cp = pltpu.CompilerParams()
for _f, _v in (("needs_layout_passes", False), ("use_tc_tiling_on_sc", False)):
    if _f in pltpu.CompilerParams.__dataclass_fields__:
        _cp = dataclasses.replace(_cp, **{_f: _v})


W1 = 512
ROWS1 = NNZ // W1
FT1 = ROWS1 // NW
REM1 = ROWS1 - FT1 * NW


def _worker_ids():
    cid = lax.axis_index("c")
    sid = lax.axis_index("s")
    gwid = sid * NC + cid
    return cid, sid, gwid


def _bulk_load(pairs, cbase, gwid, ft, rem, sems):
    @pl.when(gwid < rem)
    def _():
        ds = [pltpu.async_copy(hbm.at[pl.ds(cbase, ft + 1)], vmem, sem)
              for (hbm, vmem), sem in zip(pairs, sems)]
        for d in ds:
            d.wait()

    @pl.when(gwid >= rem)
    def _():
        ds = [pltpu.async_copy(hbm.at[pl.ds(cbase, ft)],
                               vmem.at[pl.ds(0, ft)], sem)
              for (hbm, vmem), sem in zip(pairs, sems)]
        for d in ds:
            d.wait()


@functools.partial(
    pl.kernel,
    mesh=_mesh,
    compiler_params=_cp,
    out_type=jax.ShapeDtypeStruct((NC * ND,), jnp.float32),
    scratch_types=[
        pltpu.VMEM((FT1 + 1, W1), jnp.int32),
        pltpu.VMEM((FT1 + 1, W1), jnp.float32),
        pltpu.VMEM_SHARED((ND,), jnp.float32),
        pltpu.SemaphoreType.DMA,
        pltpu.SemaphoreType.DMA,
        pltpu.SemaphoreType.DMA,
    ],
)
def _stage1(xf_hbm, xv_hbm, z_hbm, out_hbm, idx_all, vals_all, xd_sh,
            s0, s1, s2):
    cid, sid, gwid = _worker_ids()
    cbase = gwid * FT1 + jnp.minimum(gwid, REM1)

    dz = pltpu.async_copy(z_hbm, xd_sh.at[pl.ds(sid * SL1, SL1)], s1)
    _bulk_load([(xf_hbm, idx_all), (xv_hbm, vals_all)], cbase, gwid,
               FT1, REM1, (s0, s2))
    dz.wait()
    plsc.subcore_barrier()

    @pl.loop(0, FT1)
    def _(t):
        pltpu.sync_copy(vals_all.at[t], xd_sh.at[idx_all.at[t]], add=True)

    @pl.when(gwid < REM1)
    def _():
        pltpu.sync_copy(vals_all.at[FT1], xd_sh.at[idx_all.at[FT1]],
                        add=True)

    plsc.subcore_barrier()
    pltpu.sync_copy(xd_sh.at[pl.ds(sid * SL1, SL1)],
                    out_hbm.at[pl.ds(cid * ND + sid * SL1, SL1)])


@functools.partial(
    pl.kernel,
    mesh=_mesh,
    compiler_params=_cp,
    out_type=jax.ShapeDtypeStruct((NC * N, OUT), jnp.float32),
    scratch_types=[
        pltpu.VMEM((FULL_T + 1, 128), jnp.int32),
        pltpu.VMEM((FULL_T + 1, 128), jnp.int32),
        pltpu.VMEM((FULL_T + 1, 128), jnp.float32),
        pltpu.VMEM((3, 128, OUT), jnp.float32),
        pltpu.VMEM_SHARED((N, OUT), jnp.float32),
        pltpu.SemaphoreType.DMA,
        pltpu.SemaphoreType.DMA,
        pltpu.SemaphoreType.DMA,
        pltpu.SemaphoreType.DMA,
        pltpu.SemaphoreType.DMA,
        pltpu.SemaphoreType.DMA,
    ],
)
def _stage2(ar_hbm, ac_hbm, av_hbm, h_hbm, z_hbm, out_hbm,
            rows_all, cols_all, vals_all, bufs, y_sh,
            g0, g1, g2, s0, s1, s2):
    cid, sid, gwid = _worker_ids()
    cbase = gwid * FULL_T + jnp.minimum(gwid, REM)

    @pl.when(sid < 15)
    def _():
        pltpu.async_copy(z_hbm.at[pl.ds(0, 624)],
                         y_sh.at[pl.ds(sid * 624, 624)], s0)

    @pl.when(sid == 15)
    def _():
        pltpu.async_copy(z_hbm, y_sh.at[pl.ds(15 * 624, 640)], s0)

    _bulk_load([(ar_hbm, rows_all), (ac_hbm, cols_all), (av_hbm, vals_all)],
               cbase, gwid, FULL_T, REM, (g0, g1, g2))

    @pl.when(sid < 15)
    def _():
        pltpu.make_async_copy(z_hbm.at[pl.ds(0, 624)],
                              y_sh.at[pl.ds(sid * 624, 624)], s0).wait()

    @pl.when(sid == 15)
    def _():
        pltpu.make_async_copy(z_hbm, y_sh.at[pl.ds(15 * 624, 640)], s0).wait()

    plsc.subcore_barrier()

    def scale(t, b):
        @pl.loop(0, 8)
        def _(g):
            v16 = vals_all[t, pl.ds(g * 16, 16)]
            for j in range(16):
                k = g * 16 + j
                bc = jnp.full((16,), v16[j], jnp.float32)
                bufs[b, k, pl.ds(0, 16)] = bufs[b, k, pl.ds(0, 16)] * bc
                bufs[b, k, pl.ds(16, 16)] = bufs[b, k, pl.ds(16, 16)] * bc

    nchunks = jnp.where(gwid < REM, FULL_T + 1, FULL_T)
    gsem = (g0, g1, g2)
    ssem = (s0, s1, s2)

    pltpu.async_copy(h_hbm.at[cols_all.at[0]], bufs.at[0], g0)
    pltpu.async_copy(h_hbm.at[cols_all.at[1]], bufs.at[1], g1)

    def chunk_body(c, b):
        bn = (b + 2) % 3
        pltpu.make_async_copy(h_hbm.at[cols_all.at[c]], bufs.at[b],
                              gsem[b]).wait()
        scale(c, b)

        @pl.when(c > 0)
        def _():
            pltpu.make_async_copy(bufs.at[bn], y_sh.at[rows_all.at[c - 1]],
                                  ssem[bn]).wait()

        @pl.when(c + 2 < nchunks)
        def _():
            pltpu.async_copy(h_hbm.at[cols_all.at[c + 2]], bufs.at[bn],
                             gsem[bn])

        pltpu.async_copy(bufs.at[b], y_sh.at[rows_all.at[c]], ssem[b],
                         add=True)

    @pl.loop(0, FULL_T, step=3)
    def _(t):
        chunk_body(t, 0)
        chunk_body(t + 1, 1)
        chunk_body(t + 2, 2)

    pltpu.make_async_copy(bufs.at[2], y_sh.at[rows_all.at[FULL_T - 1]],
                          ssem[2]).wait()

    @pl.when(gwid < REM)
    def _():
        pltpu.make_async_copy(h_hbm.at[cols_all.at[FULL_T]], bufs.at[0],
                              gsem[0]).wait()
        scale(FULL_T, 0)
        pltpu.sync_copy(bufs.at[0], y_sh.at[rows_all.at[FULL_T]], add=True)

    plsc.subcore_barrier()

    @pl.when(sid < 15)
    def _():
        pltpu.sync_copy(y_sh.at[pl.ds(sid * 624, 624)],
                        out_hbm.at[pl.ds(cid * N + sid * 624, 624)])

    @pl.when(sid == 15)
    def _():
        pltpu.sync_copy(y_sh.at[pl.ds(15 * 624, 640)],
                        out_hbm.at[pl.ds(cid * N + 15 * 624, 640)])


def _mm_body(x0_ref, x1_ref, w_ref, h_ref):
    bn = h_ref.shape[0]
    x = (x0_ref[...] + x1_ref[...]).reshape(bn, D)
    h_ref[...] = jnp.dot(x, w_ref[...], preferred_element_type=jnp.float32)


def _matmul(xp, w):
    bn = 2000
    nb = N // bn
    return pl.pallas_call(
        _mm_body,
        grid=(nb,),
        in_specs=[
            pl.BlockSpec((bn * D,), lambda i: (i,)),
            pl.BlockSpec((bn * D,), lambda i: (i + nb,)),
            pl.BlockSpec((D, OUT), lambda i: (0, 0)),
        ],
        out_specs=pl.BlockSpec((bn, OUT), lambda i: (i, 0)),
        out_shape=jax.ShapeDtypeStruct((N, OUT), jnp.float32),
    )(xp, xp, w)


def _fin_body(y0_ref, y1_ref, o_ref):
    o_ref[...] = jnp.maximum(y0_ref[...] + y1_ref[...], 0.0)


def _finish(yp):
    bn = 2000
    nb = N // bn
    return pl.pallas_call(
        _fin_body,
        grid=(nb,),
        in_specs=[
            pl.BlockSpec((bn, OUT), lambda i: (i, 0)),
            pl.BlockSpec((bn, OUT), lambda i: (i + nb, 0)),
        ],
        out_specs=pl.BlockSpec((bn, OUT), lambda i: (i, 0)),
        out_shape=jax.ShapeDtypeStruct((N, OUT), jnp.float32),
    )(yp, yp)


def kernel(x_rows, x_cols, x_vals, adj_rows, adj_cols, adj_vals, kernel):
    xf = (x_rows.astype(jnp.int32) * D
          + x_cols.astype(jnp.int32)).reshape(ROWS1, W1)
    xv = x_vals.reshape(ROWS1, W1)
    ar = adj_rows.astype(jnp.int32).reshape(ROWS, 128)
    ac = adj_cols.astype(jnp.int32).reshape(ROWS, 128)
    av = adj_vals.reshape(ROWS, 128)
    z1 = jnp.zeros((SL1,), jnp.float32)
    z2 = jnp.zeros((640, OUT), jnp.float32)

    xd = _stage1(xf, xv, z1)
    h = _matmul(xd, kernel)
    yp = _stage2(ar, ac, av, h, z2)
    return _finish(yp)
